# baseline (speedup 1.0000x reference)
import functools

import jax
import jax.numpy as jnp
from jax import lax
from jax.experimental import pallas as pl
from jax.experimental.pallas import tpu as pltpu
from jax.experimental.pallas import tpu_sc as plsc

MOD = 100
NB, NS_SEQ = 16384, 26
N_TOTAL = NB * NS_SEQ
NC, NSC, LANES = 2, 16, 16
NW = NC * NSC
B_PER_W = NB // NW
PER_W = B_PER_W * NS_SEQ
BCHUNK = 32
CHUNK = BCHUNK * NS_SEQ
N_CHUNKS = B_PER_W // BCHUNK
D = 16
QROWS = 10001

assert NB % NW == 0 and B_PER_W % BCHUNK == 0 and N_CHUNKS % 2 == 0

_D_IOTA = None


def _sc_lookup_body(xt_hbm, qt_hbm, rt_hbm, out_hbm,
                    x_v, qi_v, ri_v, qr0, rr0, qr1, rr1, o0, o1,
                    qt_sh, rt_sh,
                    g0, g1, s0, s1):
    wid = lax.axis_index("s") * NC + lax.axis_index("c")
    b0 = wid * B_PER_W

    @pl.when(lax.axis_index("s") == 0)
    def _():
        pltpu.sync_copy(qt_hbm, qt_sh)
        pltpu.sync_copy(rt_hbm, rt_sh)

    pltpu.sync_copy(xt_hbm.at[:, pl.ds(b0, B_PER_W)], x_v)

    def divmod_s(s, carry):
        def divmod_m(m, carry2):
            v = x_v[s, pl.ds(m * LANES, LANES)]
            xf = v.astype(jnp.float32)
            q = ((xf + 0.5) * 0.01).astype(jnp.int32)
            c, cm = lax.div(m * LANES, BCHUNK), lax.rem(m * LANES, BCHUNK)
            off = c * CHUNK + s * BCHUNK + cm
            qi_v[pl.ds(off, LANES)] = q
            ri_v[pl.ds(off, LANES)] = v - q * MOD
            return carry2
        lax.fori_loop(0, B_PER_W // LANES, divmod_m, 0)
        return carry

    lax.fori_loop(0, NS_SEQ, divmod_s, 0)
    plsc.subcore_barrier()

    def start_gather(j, qr, rr, gsem):
        pltpu.make_async_copy(
            qt_sh.at[qi_v.at[pl.ds(j * CHUNK, CHUNK)]], qr, gsem).start()
        pltpu.make_async_copy(
            rt_sh.at[ri_v.at[pl.ds(j * CHUNK, CHUNK)]], rr, gsem).start()

    def wait_gather(qr, rr, gsem):
        pltpu.make_async_copy(
            qt_sh.at[qi_v.at[pl.ds(0, CHUNK)]], qr, gsem).wait()
        pltpu.make_async_copy(
            rt_sh.at[ri_v.at[pl.ds(0, CHUNK)]], rr, gsem).wait()

    def out_copy(j, o, osem):
        b_abs = b0 + j * BCHUNK
        blk = lax.div(b_abs, 128)
        off = lax.rem(b_abs, 128)
        return pltpu.make_async_copy(
            o, out_hbm.at[:, :, blk, :, pl.ds(off, BCHUNK)], osem)

    d_iota = lax.iota(jnp.int32, LANES)
    dhi_vec = lax.shift_right_logical(d_iota, 3)
    dlo_vec = lax.rem(d_iota, 8)
    bb_splats = [jnp.full((LANES,), bb, jnp.int32) for bb in range(BCHUNK)]

    def mul(qr, rr, o):
        def body(s, carry):
            s_vec = jnp.full((LANES,), s, jnp.int32)
            for bb in range(BCHUNK):
                j = s * BCHUNK + bb
                plsc.store_scatter(o, [s_vec, dhi_vec, dlo_vec,
                                       bb_splats[bb]],
                                   qr[j, :] * rr[j, :])
            return carry
        lax.fori_loop(0, NS_SEQ, body, 0)

    start_gather(0, qr0, rr0, g0)

    def step(t, carry):
        j0 = 2 * t
        start_gather(j0 + 1, qr1, rr1, g1)
        wait_gather(qr0, rr0, g0)

        @pl.when(t > 0)
        def _():
            out_copy(0, o0, s0).wait()

        mul(qr0, rr0, o0)
        out_copy(j0, o0, s0).start()

        @pl.when(t < N_CHUNKS // 2 - 1)
        def _():
            start_gather(j0 + 2, qr0, rr0, g0)

        wait_gather(qr1, rr1, g1)

        @pl.when(t > 0)
        def _():
            out_copy(0, o1, s1).wait()

        mul(qr1, rr1, o1)
        out_copy(j0 + 1, o1, s1).start()
        return carry

    lax.fori_loop(0, N_CHUNKS // 2, step, 0)
    out_copy(0, o0, s0).wait()
    out_copy(0, o1, s1).wait()


@functools.partial(
    pl.kernel,
    out_type=jax.ShapeDtypeStruct((NS_SEQ, D // 8, NB // 128, 8, 128),
                                  jnp.float32),
    mesh=plsc.VectorSubcoreMesh(core_axis_name="c", subcore_axis_name="s"),
    compiler_params=pltpu.CompilerParams(use_tc_tiling_on_sc=False,
                                         needs_layout_passes=False),
    scratch_types=[
        pltpu.VMEM((NS_SEQ, B_PER_W), jnp.int32),
        pltpu.VMEM((PER_W,), jnp.int32),
        pltpu.VMEM((PER_W,), jnp.int32),
        pltpu.VMEM((CHUNK, D), jnp.float32),
        pltpu.VMEM((CHUNK, D), jnp.float32),
        pltpu.VMEM((CHUNK, D), jnp.float32),
        pltpu.VMEM((CHUNK, D), jnp.float32),
        pltpu.VMEM((NS_SEQ, D // 8, 8, BCHUNK), jnp.float32),
        pltpu.VMEM((NS_SEQ, D // 8, 8, BCHUNK), jnp.float32),
        pltpu.VMEM_SHARED((QROWS, D), jnp.float32),
        pltpu.VMEM_SHARED((MOD, D), jnp.float32),
        pltpu.SemaphoreType.DMA,
        pltpu.SemaphoreType.DMA,
        pltpu.SemaphoreType.DMA,
        pltpu.SemaphoreType.DMA,
    ],
)
def _sc_lookup(*refs):
    _sc_lookup_body(*refs)


def kernel(x, quotient_table, remainder_table):
    out5 = _sc_lookup(x.T.astype(jnp.int32), quotient_table,
                      remainder_table)
    return out5.transpose(2, 4, 0, 1, 3).reshape(NB, NS_SEQ, D)


# speedup vs baseline: 1.7096x; 1.7096x over previous
import functools

import jax
import jax.numpy as jnp
from jax import lax
from jax.experimental import pallas as pl
from jax.experimental.pallas import tpu as pltpu
from jax.experimental.pallas import tpu_sc as plsc

MOD = 100
NB, NS_SEQ = 16384, 26
N_TOTAL = NB * NS_SEQ
NC, NSC, LANES = 2, 16, 16
NW = NC * NSC
B_PER_W = NB // NW
PER_W = B_PER_W * NS_SEQ
BCHUNK = 16
CHUNK = BCHUNK * NS_SEQ
N_CHUNKS = B_PER_W // BCHUNK
D = 16
QROWS = 10001

assert NB % NW == 0 and B_PER_W % BCHUNK == 0 and N_CHUNKS % 2 == 0

_D_IOTA = None


def _sc_lookup_body(xt_hbm, qt_hbm, rt_hbm, out_hbm,
                    x_v, qi_v, ri_v, qr0, rr0, qr1, rr1, o0, o1,
                    qt_sh, rt_sh,
                    g0, g1, s0, s1):
    wid = lax.axis_index("s") * NC + lax.axis_index("c")
    b0 = wid * B_PER_W

    @pl.when(lax.axis_index("s") == 0)
    def _():
        pltpu.sync_copy(qt_hbm, qt_sh)
        pltpu.sync_copy(rt_hbm, rt_sh)

    pltpu.sync_copy(xt_hbm.at[:, pl.ds(b0, B_PER_W)], x_v)

    def divmod_s(s, carry):
        def divmod_m(m, carry2):
            v = x_v[s, pl.ds(m * LANES, LANES)]
            xf = v.astype(jnp.float32)
            q = ((xf + 0.5) * 0.01).astype(jnp.int32)
            off = m * CHUNK + s * LANES
            qi_v[pl.ds(off, LANES)] = q
            ri_v[pl.ds(off, LANES)] = v - q * MOD
            return carry2
        lax.fori_loop(0, N_CHUNKS, divmod_m, 0)
        return carry

    lax.fori_loop(0, NS_SEQ, divmod_s, 0)
    plsc.subcore_barrier()

    def start_gather(j, qr, rr, gsem):
        pltpu.make_async_copy(
            qt_sh.at[qi_v.at[pl.ds(j * CHUNK, CHUNK)]], qr, gsem).start()
        pltpu.make_async_copy(
            rt_sh.at[ri_v.at[pl.ds(j * CHUNK, CHUNK)]], rr, gsem).start()

    def wait_gather(qr, rr, gsem):
        pltpu.make_async_copy(
            qt_sh.at[qi_v.at[pl.ds(0, CHUNK)]], qr, gsem).wait()
        pltpu.make_async_copy(
            rt_sh.at[ri_v.at[pl.ds(0, CHUNK)]], rr, gsem).wait()

    def out_copy(j, o, osem):
        b_abs = b0 + j * BCHUNK
        blk = lax.div(b_abs, 128)
        off = lax.rem(b_abs, 128)
        return pltpu.make_async_copy(
            o, out_hbm.at[:, :, blk, :, pl.ds(off, BCHUNK)], osem)

    d_iota = lax.iota(jnp.int32, LANES)
    dhi_vec = lax.shift_right_logical(d_iota, 3)
    dlo_vec = lax.rem(d_iota, 8)
    bb_splats = [jnp.full((LANES,), bb, jnp.int32) for bb in range(BCHUNK)]

    def mul(qr, rr, o):
        def body(s, carry):
            s_vec = jnp.full((LANES,), s, jnp.int32)
            for bb in range(BCHUNK):
                j = s * BCHUNK + bb
                plsc.store_scatter(o, [s_vec, dhi_vec, dlo_vec,
                                       bb_splats[bb]],
                                   qr[j, :] * rr[j, :])
            return carry
        lax.fori_loop(0, NS_SEQ, body, 0)

    start_gather(0, qr0, rr0, g0)

    def step(t, carry):
        j0 = 2 * t
        start_gather(j0 + 1, qr1, rr1, g1)
        wait_gather(qr0, rr0, g0)

        @pl.when(t > 0)
        def _():
            out_copy(0, o0, s0).wait()

        mul(qr0, rr0, o0)
        out_copy(j0, o0, s0).start()

        @pl.when(t < N_CHUNKS // 2 - 1)
        def _():
            start_gather(j0 + 2, qr0, rr0, g0)

        wait_gather(qr1, rr1, g1)

        @pl.when(t > 0)
        def _():
            out_copy(0, o1, s1).wait()

        mul(qr1, rr1, o1)
        out_copy(j0 + 1, o1, s1).start()
        return carry

    lax.fori_loop(0, N_CHUNKS // 2, step, 0)
    out_copy(0, o0, s0).wait()
    out_copy(0, o1, s1).wait()


@functools.partial(
    pl.kernel,
    out_type=jax.ShapeDtypeStruct((NS_SEQ, D // 8, NB // 128, 8, 128),
                                  jnp.float32),
    mesh=plsc.VectorSubcoreMesh(core_axis_name="c", subcore_axis_name="s"),
    compiler_params=pltpu.CompilerParams(use_tc_tiling_on_sc=False,
                                         needs_layout_passes=False),
    scratch_types=[
        pltpu.VMEM((NS_SEQ, B_PER_W), jnp.int32),
        pltpu.VMEM((PER_W,), jnp.int32),
        pltpu.VMEM((PER_W,), jnp.int32),
        pltpu.VMEM((CHUNK, D), jnp.float32),
        pltpu.VMEM((CHUNK, D), jnp.float32),
        pltpu.VMEM((CHUNK, D), jnp.float32),
        pltpu.VMEM((CHUNK, D), jnp.float32),
        pltpu.VMEM((NS_SEQ, D // 8, 8, BCHUNK), jnp.float32),
        pltpu.VMEM((NS_SEQ, D // 8, 8, BCHUNK), jnp.float32),
        pltpu.VMEM_SHARED((QROWS, D), jnp.float32),
        pltpu.VMEM_SHARED((MOD, D), jnp.float32),
        pltpu.SemaphoreType.DMA,
        pltpu.SemaphoreType.DMA,
        pltpu.SemaphoreType.DMA,
        pltpu.SemaphoreType.DMA,
    ],
)
def _sc_lookup(*refs):
    _sc_lookup_body(*refs)


def kernel(x, quotient_table, remainder_table):
    out5 = _sc_lookup(x.T.astype(jnp.int32), quotient_table,
                      remainder_table)
    return out5.transpose(2, 4, 0, 1, 3).reshape(NB, NS_SEQ, D)
